# baseline (speedup 1.0000x reference)
import functools

import jax
import jax.numpy as jnp
from jax import lax
from jax.experimental import pallas as pl
from jax.experimental.pallas import tpu as pltpu
from jax.experimental.pallas import tpu_sc as plsc

S_LEN = 200
BATCH = 1024
N_TOK = S_LEN * BATCH
D_WORD = 64
D_TAG = 16
D_CROSS = 2 * D_TAG
D_OUT = D_WORD + D_CROSS
POS_DICT = 50
NER_DICT = 20
VOCAB = 1000000

NUM_CORES = 2
NUM_SUBCORES = 16
NW = NUM_CORES * NUM_SUBCORES
TOK_PER_W = N_TOK // NW
SUB = 128
NCH = TOK_PER_W // SUB
RING = 6
DEPTH = 4
LANES = 16
FB = D_OUT // 8
BB = BATCH // SUB

TW = 3200


def _tc_transpose_table(wt_t):
    ident = jnp.eye(D_WORD, dtype=jnp.float32)

    def body(x_ref, i_ref, o_ref):
        x = x_ref[...]
        o_ref[...] = lax.dot_general(
            x, i_ref[...], (((0,), (0,)), ((), ())),
            preferred_element_type=jnp.float32,
        )

    grid = (VOCAB + TW - 1) // TW
    return pl.pallas_call(
        body,
        grid=(grid,),
        in_specs=[
            pl.BlockSpec((D_WORD, TW), lambda i: (0, i)),
            pl.BlockSpec((D_WORD, D_WORD), lambda i: (0, 0)),
        ],
        out_specs=pl.BlockSpec((TW, D_WORD), lambda i: (i, 0)),
        out_shape=jax.ShapeDtypeStruct((VOCAB, D_WORD), jnp.float32),
    )(wt_t, ident)


def _tc_pack_out(out2d):
    ident = jnp.eye(SUB, dtype=jnp.float32)

    def body(x_ref, i_ref, o_ref):
        x = x_ref[...]
        y = lax.dot_general(
            x, i_ref[...], (((0,), (0,)), ((), ())),
            preferred_element_type=jnp.float32,
        )
        o_ref[...] = y.reshape(1, FB, 1, 8, SUB)

    return pl.pallas_call(
        body,
        grid=(S_LEN, BB),
        in_specs=[
            pl.BlockSpec((SUB, D_OUT), lambda s, b: (s * BB + b, 0)),
            pl.BlockSpec((SUB, SUB), lambda s, b: (0, 0)),
        ],
        out_specs=pl.BlockSpec(
            (1, FB, 1, 8, SUB), lambda s, b: (s, 0, b, 0, 0)
        ),
        out_shape=jax.ShapeDtypeStruct((S_LEN, FB, BB, 8, SUB), jnp.float32),
    )(out2d, ident)


def _sc_embed(word_table, cross_table, idxw, idxp, idxn):
    mesh = plsc.VectorSubcoreMesh(core_axis_name="c", subcore_axis_name="s")

    @functools.partial(
        pl.kernel,
        out_type=jax.ShapeDtypeStruct((N_TOK, D_OUT), jnp.float32),
        mesh=mesh,
        scratch_types=[
            pltpu.VMEM((NCH, SUB), jnp.int32),
            pltpu.VMEM((NCH, SUB), jnp.int32),
            pltpu.VMEM((NCH, SUB), jnp.int32),
            pltpu.VMEM((NCH, SUB), jnp.int32),
            pltpu.VMEM((RING, SUB, D_WORD), jnp.float32),
            pltpu.VMEM((RING, SUB, D_CROSS), jnp.float32),
            pltpu.SemaphoreType.DMA,
            pltpu.SemaphoreType.DMA,
        ],
        compiler_params=pltpu.CompilerParams(use_tc_tiling_on_sc=False),
    )
    def k(wt, ct, iw, ip, inr, out, iw_v, ip_v, in_v, it_v, wbuf, tbuf, gsem, wsem):
        wid = lax.axis_index("s") * NUM_CORES + lax.axis_index("c")
        base_tok = wid * TOK_PER_W

        pltpu.sync_copy(iw.at[wid], iw_v)
        pltpu.sync_copy(ip.at[wid], ip_v)
        pltpu.sync_copy(inr.at[wid], in_v)

        def tag_body(r, c):
            for g in range(SUB // LANES):
                p = ip_v[r, pl.ds(g * LANES, LANES)]
                n = in_v[r, pl.ds(g * LANES, LANES)]
                it_v[r, pl.ds(g * LANES, LANES)] = p * NER_DICT + n
            return c

        lax.fori_loop(0, NCH, tag_body, 0)

        def fire(cg, slot):
            pltpu.make_async_copy(wt.at[iw_v.at[cg]], wbuf.at[slot], gsem).start()
            pltpu.make_async_copy(ct.at[it_v.at[cg]], tbuf.at[slot], gsem).start()

        def write_descs(slot, t0):
            return (
                pltpu.make_async_copy(
                    wbuf.at[slot], out.at[pl.ds(t0, SUB), pl.ds(0, D_WORD)], wsem
                ),
                pltpu.make_async_copy(
                    tbuf.at[slot], out.at[pl.ds(t0, SUB), pl.ds(D_WORD, D_CROSS)], wsem
                ),
            )

        for cg in range(DEPTH):
            fire(cg, cg)

        def body(ci, c):
            cg = ci + DEPTH
            slot_g = lax.rem(cg, RING)

            @pl.when(jnp.logical_and(cg < NCH, cg >= RING))
            def _():
                for d in write_descs(slot_g, base_tok):
                    d.wait()

            @pl.when(cg < NCH)
            def _():
                fire(cg, slot_g)

            slot = lax.rem(ci, RING)
            t0 = base_tok + ci * SUB
            pltpu.make_async_copy(wt.at[iw_v.at[ci]], wbuf.at[slot], gsem).wait()
            pltpu.make_async_copy(ct.at[it_v.at[ci]], tbuf.at[slot], gsem).wait()
            for d in write_descs(slot, t0):
                d.start()
            return c

        lax.fori_loop(0, NCH, body, 0)

        for _i in range(RING):
            for d in write_descs(0, base_tok):
                d.wait()

    return k(word_table, cross_table, idxw, idxp, idxn)


def kernel(seq_word, seq_pos, seq_ner, word_table, pos_table, ner_table):
    cross = jnp.concatenate(
        [
            jnp.broadcast_to(pos_table[:, None, :], (POS_DICT, NER_DICT, D_TAG)),
            jnp.broadcast_to(ner_table[None, :, :], (POS_DICT, NER_DICT, D_TAG)),
        ],
        axis=2,
    ).reshape(POS_DICT * NER_DICT, D_CROSS)
    wt_rm = _tc_transpose_table(word_table.T)
    idxw = seq_word.reshape(NW, NCH, SUB).astype(jnp.int32)
    idxp = seq_pos.reshape(NW, NCH, SUB).astype(jnp.int32)
    idxn = seq_ner.reshape(NW, NCH, SUB).astype(jnp.int32)
    out2d = _sc_embed(wt_rm, cross, idxw, idxp, idxn)
    out5 = _tc_pack_out(out2d)
    return out5.transpose(0, 2, 4, 1, 3).reshape(S_LEN, BATCH, D_OUT)


# speedup vs baseline: 2.2229x; 2.2229x over previous
import functools

import jax
import jax.numpy as jnp
from jax import lax
from jax.experimental import pallas as pl
from jax.experimental.pallas import tpu as pltpu
from jax.experimental.pallas import tpu_sc as plsc

S_LEN = 200
BATCH = 1024
N_TOK = S_LEN * BATCH
D_WORD = 64
D_TAG = 16
D_CROSS = 2 * D_TAG
D_OUT = D_WORD + D_CROSS
POS_DICT = 50
NER_DICT = 20

NUM_CORES = 2
NUM_SUBCORES = 16
NW = NUM_CORES * NUM_SUBCORES
TOK_PER_W = N_TOK // NW
SUB = 128
NCH = TOK_PER_W // SUB
RING = 6
DEPTH = 4
LANES = 16


def _sc_embed(word_table, cross_table, idxw, idxp, idxn):
    mesh = plsc.VectorSubcoreMesh(core_axis_name="c", subcore_axis_name="s")

    @functools.partial(
        pl.kernel,
        out_type=jax.ShapeDtypeStruct((N_TOK, D_OUT), jnp.float32),
        mesh=mesh,
        scratch_types=[
            pltpu.VMEM((NCH, SUB), jnp.int32),
            pltpu.VMEM((NCH, SUB), jnp.int32),
            pltpu.VMEM((NCH, SUB), jnp.int32),
            pltpu.VMEM((NCH, SUB), jnp.int32),
            pltpu.VMEM((RING, SUB, D_WORD), jnp.float32),
            pltpu.VMEM((RING, SUB, D_CROSS), jnp.float32),
            pltpu.SemaphoreType.DMA,
            pltpu.SemaphoreType.DMA,
        ],
        compiler_params=pltpu.CompilerParams(
            use_tc_tiling_on_sc=False, disable_bounds_checks=True
        ),
    )
    def k(wt, ct, iw, ip, inr, out, iw_v, ip_v, in_v, it_v, wbuf, tbuf, gsem, wsem):
        wid = lax.axis_index("s") * NUM_CORES + lax.axis_index("c")
        base_tok = wid * TOK_PER_W

        pltpu.sync_copy(iw.at[wid], iw_v)
        pltpu.sync_copy(ip.at[wid], ip_v)
        pltpu.sync_copy(inr.at[wid], in_v)

        def tag_body(r, c):
            for g in range(SUB // LANES):
                p = ip_v[r, pl.ds(g * LANES, LANES)]
                n = in_v[r, pl.ds(g * LANES, LANES)]
                it_v[r, pl.ds(g * LANES, LANES)] = p * NER_DICT + n
            return c

        lax.fori_loop(0, NCH, tag_body, 0)

        def fire(cg, slot):
            pltpu.make_async_copy(wt.at[iw_v.at[cg]], wbuf.at[slot], gsem).start()
            pltpu.make_async_copy(ct.at[it_v.at[cg]], tbuf.at[slot], gsem).start()

        def write_descs(slot, t0):
            return (
                pltpu.make_async_copy(
                    wbuf.at[slot], out.at[pl.ds(t0, SUB), pl.ds(0, D_WORD)], wsem
                ),
                pltpu.make_async_copy(
                    tbuf.at[slot], out.at[pl.ds(t0, SUB), pl.ds(D_WORD, D_CROSS)], wsem
                ),
            )

        for cg in range(DEPTH):
            fire(cg, cg)

        def body(ci, c):
            cg = ci + DEPTH
            slot_g = lax.rem(cg, RING)

            @pl.when(jnp.logical_and(cg < NCH, cg >= RING))
            def _():
                for d in write_descs(slot_g, base_tok):
                    d.wait()

            @pl.when(cg < NCH)
            def _():
                fire(cg, slot_g)

            slot = lax.rem(ci, RING)
            t0 = base_tok + ci * SUB
            pltpu.make_async_copy(wt.at[iw_v.at[ci]], wbuf.at[slot], gsem).wait()
            pltpu.make_async_copy(ct.at[it_v.at[ci]], tbuf.at[slot], gsem).wait()
            for d in write_descs(slot, t0):
                d.start()
            return c

        lax.fori_loop(0, NCH, body, 0)

        for _i in range(RING):
            for d in write_descs(0, base_tok):
                d.wait()

    return k(word_table, cross_table, idxw, idxp, idxn)


def kernel(seq_word, seq_pos, seq_ner, word_table, pos_table, ner_table):
    cross = jnp.concatenate(
        [
            jnp.broadcast_to(pos_table[:, None, :], (POS_DICT, NER_DICT, D_TAG)),
            jnp.broadcast_to(ner_table[None, :, :], (POS_DICT, NER_DICT, D_TAG)),
        ],
        axis=2,
    ).reshape(POS_DICT * NER_DICT, D_CROSS)
    idxw = seq_word.reshape(NW, NCH, SUB).astype(jnp.int32)
    idxp = seq_pos.reshape(NW, NCH, SUB).astype(jnp.int32)
    idxn = seq_ner.reshape(NW, NCH, SUB).astype(jnp.int32)
    out = _sc_embed(word_table, cross, idxw, idxp, idxn)
    return out.reshape(S_LEN, BATCH, D_OUT)


# speedup vs baseline: 2.2237x; 1.0004x over previous
import functools

import jax
import jax.numpy as jnp
from jax import lax
from jax.experimental import pallas as pl
from jax.experimental.pallas import tpu as pltpu
from jax.experimental.pallas import tpu_sc as plsc

S_LEN = 200
BATCH = 1024
N_TOK = S_LEN * BATCH
D_WORD = 64
D_TAG = 16
D_CROSS = 2 * D_TAG
D_OUT = D_WORD + D_CROSS
POS_DICT = 50
NER_DICT = 20

NUM_CORES = 2
NUM_SUBCORES = 16
NW = NUM_CORES * NUM_SUBCORES
TOK_PER_W = N_TOK // NW
SUB = 128
NCH = TOK_PER_W // SUB
RING = 8
DEPTH = 6
LANES = 16


def _sc_embed(word_table, cross_table, idxw, idxp, idxn):
    mesh = plsc.VectorSubcoreMesh(core_axis_name="c", subcore_axis_name="s")

    @functools.partial(
        pl.kernel,
        out_type=jax.ShapeDtypeStruct((N_TOK, D_OUT), jnp.float32),
        mesh=mesh,
        scratch_types=[
            pltpu.VMEM((NCH, SUB), jnp.int32),
            pltpu.VMEM((NCH, SUB), jnp.int32),
            pltpu.VMEM((NCH, SUB), jnp.int32),
            pltpu.VMEM((NCH, SUB), jnp.int32),
            pltpu.VMEM((RING, SUB, D_WORD), jnp.float32),
            pltpu.VMEM((RING, SUB, D_CROSS), jnp.float32),
            pltpu.SemaphoreType.DMA,
            pltpu.SemaphoreType.DMA,
        ],
        compiler_params=pltpu.CompilerParams(
            use_tc_tiling_on_sc=False, disable_bounds_checks=True,
            skip_device_barrier=True
        ),
    )
    def k(wt, ct, iw, ip, inr, out, iw_v, ip_v, in_v, it_v, wbuf, tbuf, gsem, wsem):
        wid = lax.axis_index("s") * NUM_CORES + lax.axis_index("c")
        base_tok = wid * TOK_PER_W

        pltpu.sync_copy(iw.at[wid], iw_v)
        pltpu.sync_copy(ip.at[wid], ip_v)
        pltpu.sync_copy(inr.at[wid], in_v)

        def tag_body(r, c):
            for g in range(SUB // LANES):
                p = ip_v[r, pl.ds(g * LANES, LANES)]
                n = in_v[r, pl.ds(g * LANES, LANES)]
                it_v[r, pl.ds(g * LANES, LANES)] = p * NER_DICT + n
            return c

        lax.fori_loop(0, NCH, tag_body, 0)

        def fire(cg, slot):
            pltpu.make_async_copy(wt.at[iw_v.at[cg]], wbuf.at[slot], gsem).start()
            pltpu.make_async_copy(ct.at[it_v.at[cg]], tbuf.at[slot], gsem).start()

        def write_descs(slot, t0):
            return (
                pltpu.make_async_copy(
                    wbuf.at[slot], out.at[pl.ds(t0, SUB), pl.ds(0, D_WORD)], wsem
                ),
                pltpu.make_async_copy(
                    tbuf.at[slot], out.at[pl.ds(t0, SUB), pl.ds(D_WORD, D_CROSS)], wsem
                ),
            )

        for cg in range(DEPTH):
            fire(cg, cg)

        def body(ci, c):
            cg = ci + DEPTH
            slot_g = lax.rem(cg, RING)

            @pl.when(jnp.logical_and(cg < NCH, cg >= RING))
            def _():
                for d in write_descs(slot_g, base_tok):
                    d.wait()

            @pl.when(cg < NCH)
            def _():
                fire(cg, slot_g)

            slot = lax.rem(ci, RING)
            t0 = base_tok + ci * SUB
            pltpu.make_async_copy(wt.at[iw_v.at[ci]], wbuf.at[slot], gsem).wait()
            pltpu.make_async_copy(ct.at[it_v.at[ci]], tbuf.at[slot], gsem).wait()
            for d in write_descs(slot, t0):
                d.start()
            return c

        lax.fori_loop(0, NCH, body, 0)

        for _i in range(RING):
            for d in write_descs(0, base_tok):
                d.wait()

    return k(word_table, cross_table, idxw, idxp, idxn)


def kernel(seq_word, seq_pos, seq_ner, word_table, pos_table, ner_table):
    cross = jnp.concatenate(
        [
            jnp.broadcast_to(pos_table[:, None, :], (POS_DICT, NER_DICT, D_TAG)),
            jnp.broadcast_to(ner_table[None, :, :], (POS_DICT, NER_DICT, D_TAG)),
        ],
        axis=2,
    ).reshape(POS_DICT * NER_DICT, D_CROSS)
    idxw = seq_word.reshape(NW, NCH, SUB).astype(jnp.int32)
    idxp = seq_pos.reshape(NW, NCH, SUB).astype(jnp.int32)
    idxn = seq_ner.reshape(NW, NCH, SUB).astype(jnp.int32)
    out = _sc_embed(word_table, cross, idxw, idxp, idxn)
    return out.reshape(S_LEN, BATCH, D_OUT)
